# baseline (speedup 1.0000x reference)
import functools

import jax
import jax.numpy as jnp
from jax import lax
from jax.experimental import pallas as pl
from jax.experimental.pallas import tpu as pltpu
from jax.experimental.pallas import tpu_sc as plsc

_B = 16384
_L = 16
_NC = 2
_NS = 16
_NW = _NC * _NS
_CHUNK = _B // _NW
_IDXW = 128
_NIDX = _CHUNK // _IDXW
_NG = _CHUNK // _L


def _mud_body(users, items, g2, uEmbed, uBias, itemEmbed, itemBias, price,
              rmf_uE, rmf_iE, rmf_uB, rmf_iB, out,
              idxu, idxi, uE_v, iE_v, ruE_v, riE_v,
              uB_v, iB_v, ruB_v, riB_v, p_v, g_v, out_v, sem):
    wid = lax.axis_index("s") * _NC + lax.axis_index("c")
    base = pl.multiple_of(wid * _CHUNK, _CHUNK)

    for j in range(_NIDX):
        pltpu.sync_copy(users.at[pl.ds(base + j * _IDXW, _IDXW)], idxu.at[j])
        pltpu.sync_copy(items.at[pl.ds(base + j * _IDXW, _IDXW)], idxi.at[j])
    pltpu.sync_copy(g2, g_v)

    descs = []
    for j in range(_NIDX):
        s = pl.ds(j * _IDXW, _IDXW)
        iu = idxu.at[j]
        ii = idxi.at[j]
        descs.append(pltpu.async_copy(uEmbed.at[iu], uE_v.at[s], sem))
        descs.append(pltpu.async_copy(rmf_uE.at[iu], ruE_v.at[s], sem))
        descs.append(pltpu.async_copy(uBias.at[iu], uB_v.at[s], sem))
        descs.append(pltpu.async_copy(rmf_uB.at[iu], ruB_v.at[s], sem))
        descs.append(pltpu.async_copy(itemEmbed.at[ii], iE_v.at[s], sem))
        descs.append(pltpu.async_copy(rmf_iE.at[ii], riE_v.at[s], sem))
        descs.append(pltpu.async_copy(itemBias.at[ii], iB_v.at[s], sem))
        descs.append(pltpu.async_copy(rmf_iB.at[ii], riB_v.at[s], sem))
        descs.append(pltpu.async_copy(price.at[ii], p_v.at[s], sem))
    for d in descs:
        d.wait()

    lanes = lax.iota(jnp.int32, 16)
    cols = [(lanes + l) & 15 for l in range(_L)]
    gB = g_v[0]
    rg = g_v[1]

    def group(g, carry):
        row = g * _L + lanes
        acc_a = jnp.zeros((16,), jnp.float32)
        acc_r = jnp.zeros((16,), jnp.float32)
        for l in range(_L):
            c = cols[l]
            ue = plsc.load_gather(uE_v, [row, c])
            ie = plsc.load_gather(iE_v, [row, c])
            acc_a = acc_a + ue * ie
            rue = plsc.load_gather(ruE_v, [row, c])
            rie = plsc.load_gather(riE_v, [row, c])
            acc_r = acc_r + rue * rie
        s = pl.ds(pl.multiple_of(g * _L, _L), _L)
        alpha = gB + uB_v[s] + iB_v[s] + acc_a
        r = rg + ruB_v[s] + riB_v[s] + acc_r
        tanh_r = 1.0 - 2.0 / (jnp.exp(2.0 * r) + 1.0)
        inv_sig = 1.0 + jnp.exp(-p_v[s])
        out_v[s] = 0.5 * alpha * tanh_r * inv_sig
        return carry

    lax.fori_loop(0, _NG, group, 0)
    pltpu.sync_copy(out_v, out.at[pl.ds(base, _CHUNK)])


_mud_sc = functools.partial(
    pl.kernel,
    out_type=jax.ShapeDtypeStruct((_B,), jnp.float32),
    mesh=plsc.VectorSubcoreMesh(core_axis_name="c", subcore_axis_name="s"),
    compiler_params=pltpu.CompilerParams(
        needs_layout_passes=False, use_tc_tiling_on_sc=False),
    scratch_types=[
        pltpu.VMEM((_NIDX, _IDXW), jnp.int32),
        pltpu.VMEM((_NIDX, _IDXW), jnp.int32),
        pltpu.VMEM((_CHUNK, _L), jnp.float32),
        pltpu.VMEM((_CHUNK, _L), jnp.float32),
        pltpu.VMEM((_CHUNK, _L), jnp.float32),
        pltpu.VMEM((_CHUNK, _L), jnp.float32),
        pltpu.VMEM((_CHUNK,), jnp.float32),
        pltpu.VMEM((_CHUNK,), jnp.float32),
        pltpu.VMEM((_CHUNK,), jnp.float32),
        pltpu.VMEM((_CHUNK,), jnp.float32),
        pltpu.VMEM((_CHUNK,), jnp.float32),
        pltpu.VMEM((2, 16), jnp.float32),
        pltpu.VMEM((_CHUNK,), jnp.float32),
        pltpu.SemaphoreType.DMA,
    ],
)(_mud_body)


def kernel(users, items, gBias, uBias, itemBias, uEmbed, itemEmbed, price,
           rmf_uE, rmf_iE, rmf_uB, rmf_iB, rmf_g):
    users = users.astype(jnp.int32)
    items = items.astype(jnp.int32)
    g2 = jnp.concatenate([
        jnp.broadcast_to(gBias.reshape(1, 1), (1, 16)),
        jnp.broadcast_to(rmf_g.reshape(1, 1), (1, 16)),
    ], axis=0)
    return _mud_sc(users, items, g2,
                   uEmbed, uBias.reshape(-1), itemEmbed, itemBias.reshape(-1),
                   price, rmf_uE, rmf_iE, rmf_uB.reshape(-1), rmf_iB.reshape(-1))
